# baseline (speedup 1.0000x reference)
import functools

import jax
import jax.numpy as jnp
from jax import lax
from jax.experimental import pallas as pl
from jax.experimental.pallas import tpu as pltpu
from jax.experimental.pallas import tpu_sc as plsc

B, S, T, H, NUM_STATES = 8, 1000, 24, 256, 3
NC, NS, L = 2, 16, 16
NW = NC * NS
SPW = (B * S) // NW
NBLK = S // SPW
SPAD = 256
CPB = NUM_STATES * SPAD
Q = SPAD // 4
P1S = 8
P1R = NUM_STATES * P1S
P1N = Q // P1S
CS = 4
G = CS * T
P2N = SPW // CS
P2T = SPW - CS * P2N
GT = P2T * T
HS = H // L


def _sc_body(ids_hbm, state_hbm, species_hbm, out_hbm, c_hbm,
             state_v, ids_v, spc_v, comb_v, stage_v, idx_a, idx_b, pat_v,
             ssem0, ssem1, csem0, csem1, gsem0, gsem1, wsem0, wsem1):
    cid = lax.axis_index("c")
    sid = lax.axis_index("s")
    wid = sid * NC + cid
    b = wid // NBLK
    blk = wid % NBLK
    obase = (b * S + blk * SPW) * T

    pltpu.sync_copy(state_hbm, state_v)
    pltpu.sync_copy(ids_hbm.at[b, blk, 0], ids_v)

    iota = lax.iota(jnp.int32, L)
    for k in range(G // L):
        e = iota + (k * L)
        step = jnp.zeros((L,), jnp.int32)
        for m in range(1, CS):
            step = step + jnp.where(e >= m * T, 1, 0).astype(jnp.int32)
        pat_v[pl.ds(k * L, L)] = step * NUM_STATES

    ssems = (ssem0, ssem1)
    csems = (csem0, csem1)
    gsems = (gsem0, gsem1)
    wsems = (wsem0, wsem1)

    p1blk = sid // 4
    q = sid % 4
    scbase = cid * (NBLK * CPB)
    srow0 = q * Q
    crow0 = scbase + p1blk * CPB + q * NUM_STATES * Q

    def spc_read(c, buf):
        srow = pl.multiple_of(srow0 + c * P1S, P1S)
        pltpu.async_copy(
            species_hbm.at[p1blk, pl.ds(srow, P1S)], spc_v.at[buf],
            ssems[buf])

    def spc_wait(buf):
        pltpu.make_async_copy(
            species_hbm.at[p1blk, pl.ds(0, P1S)], spc_v.at[buf],
            ssems[buf]).wait()

    def c_write(c, buf):
        crow = pl.multiple_of(crow0 + c * P1R, 8)
        pltpu.async_copy(comb_v.at[buf], c_hbm.at[pl.ds(crow, P1R)],
                         csems[buf])

    def c_wait(buf):
        pltpu.make_async_copy(
            comb_v.at[buf], c_hbm.at[pl.ds(0, P1R)], csems[buf]).wait()

    def p1_compute(buf):
        for si in range(P1S):
            for j in range(NUM_STATES):
                for k in range(HS):
                    sl = pl.ds(k * L, L)
                    comb_v[buf, si * NUM_STATES + j, sl] = (
                        state_v[j, sl] + spc_v[buf, si, sl])

    spc_read(0, 0)

    def p1_body(p, carry):
        for buf in range(2):
            c = 2 * p + buf

            @pl.when(c + 1 < P1N)
            def _rd():
                spc_read(c + 1, 1 - buf)

            spc_wait(buf)

            @pl.when(p > 0)
            def _cw():
                c_wait(buf)

            p1_compute(buf)
            c_write(c, buf)
        return carry

    lax.fori_loop(0, P1N // 2, p1_body, 0)
    for buf in range(2):
        c_wait(buf)
    plsc.subcore_barrier()

    idxs = (idx_a, idx_b)
    cbase = scbase + blk * CPB

    def make_idx(ci, buf, n_e):
        base = cbase + ci * (NUM_STATES * CS)
        bvec = jnp.zeros((L,), jnp.int32) + base
        for k in range(n_e // L):
            sl = pl.ds(k * L, L)
            idxs[buf][sl] = ids_v[pl.ds(ci * G + k * L, L)] + pat_v[sl] + bvec
        for k in range(n_e // L, G // L):
            sl = pl.ds(k * L, L)
            idxs[buf][sl] = jnp.zeros((L,), jnp.int32) + cbase

    def gather(buf):
        pltpu.async_copy(c_hbm.at[idxs[buf]], stage_v.at[buf], gsems[buf])

    def gather_wait(buf):
        pltpu.make_async_copy(
            c_hbm.at[idxs[buf]], stage_v.at[buf], gsems[buf]).wait()

    def out_write(cc, buf):
        orow = pl.multiple_of(obase + cc * G, 8)
        pltpu.async_copy(
            stage_v.at[buf], out_hbm.at[pl.ds(orow, G)], wsems[buf])

    def out_wait(buf):
        pltpu.make_async_copy(
            stage_v.at[buf], out_hbm.at[pl.ds(obase, G)], wsems[buf]).wait()

    make_idx(0, 0, G)
    gather(0)

    def p2_body(p, carry):
        for buf in range(2):
            cc = 2 * p + buf
            nxt = 1 - buf

            if buf == 0:
                @pl.when(p > 0)
                def _ww():
                    out_wait(1)
            else:
                out_wait(0)

            make_idx(cc + 1, nxt, G)
            gather(nxt)
            gather_wait(buf)
            out_write(cc, buf)
        return carry

    lax.fori_loop(0, (P2N - 2) // 2, p2_body, 0)
    out_wait(1)
    make_idx(P2N - 1, 1, G)
    gather(1)
    gather_wait(0)
    out_write(P2N - 2, 0)

    out_wait(0)
    make_idx(P2N, 0, GT)
    gather(0)
    gather_wait(1)
    out_write(P2N - 1, 1)

    gather_wait(0)
    pltpu.async_copy(
        stage_v.at[0, pl.ds(0, GT)],
        out_hbm.at[pl.ds(obase + P2N * G, GT)], wsem0)
    pltpu.make_async_copy(
        stage_v.at[0, pl.ds(0, GT)], out_hbm.at[pl.ds(obase, GT)],
        wsem0).wait()
    out_wait(1)


def kernel(input_ids, state_table, species_table):
    ids4 = input_ids.reshape(B, NBLK, 1, SPW * T)
    species_pad = jnp.concatenate(
        [species_table.reshape(NBLK, SPW, H),
         jnp.zeros((NBLK, SPAD - SPW, H), jnp.float32)], axis=1)
    mesh = plsc.VectorSubcoreMesh(core_axis_name="c", subcore_axis_name="s")
    f = functools.partial(
        pl.kernel,
        mesh=mesh,
        out_type=(
            jax.ShapeDtypeStruct((B * S * T, H), jnp.float32),
            jax.ShapeDtypeStruct((NC * NBLK * CPB, H), jnp.float32),
        ),
        scratch_types=[
            pltpu.VMEM((NUM_STATES, H), jnp.float32),
            pltpu.VMEM((SPW * T,), jnp.int32),
            pltpu.VMEM((2, P1S, H), jnp.float32),
            pltpu.VMEM((2, P1R, H), jnp.float32),
            pltpu.VMEM((2, G, H), jnp.float32),
            pltpu.VMEM((G,), jnp.int32),
            pltpu.VMEM((G,), jnp.int32),
            pltpu.VMEM((G,), jnp.int32),
            pltpu.SemaphoreType.DMA,
            pltpu.SemaphoreType.DMA,
            pltpu.SemaphoreType.DMA,
            pltpu.SemaphoreType.DMA,
            pltpu.SemaphoreType.DMA,
            pltpu.SemaphoreType.DMA,
            pltpu.SemaphoreType.DMA,
            pltpu.SemaphoreType.DMA,
        ],
    )(_sc_body)
    out2, _ = f(ids4, state_table, species_pad)
    return out2.reshape(B, S, T, H)


# speedup vs baseline: 1.2857x; 1.2857x over previous
import functools

import jax
import jax.numpy as jnp
from jax import lax
from jax.experimental import pallas as pl
from jax.experimental.pallas import tpu as pltpu
from jax.experimental.pallas import tpu_sc as plsc

B, S, T, H, NUM_STATES = 8, 1000, 24, 256, 3
NC, NS, L = 2, 16, 16
NW = NC * NS
SPW = (B * S) // NW
NBLK = S // SPW
CPW = 768
P1S = 8
P1R = NUM_STATES * P1S
P1N = SPW // P1S
P1T = SPW - P1S * P1N
RT = SPW * T
GN = 128
P2N = RT // GN
GT = RT - GN * P2N
HS = H // L


def _sc_body(ids_hbm, state_hbm, species_hbm, out_hbm, c_hbm,
             state_v, ids_v, spc_v, comb_v, stage_v, idxf_v,
             ssem0, ssem1, csem0, csem1, gsem0, gsem1, wsem0, wsem1):
    cid = lax.axis_index("c")
    sid = lax.axis_index("s")
    wid = sid * NC + cid
    b = wid // NBLK
    blk = wid % NBLK
    cbase = wid * CPW
    obase = (b * S + blk * SPW) * T

    pltpu.sync_copy(state_hbm, state_v)
    pltpu.sync_copy(ids_hbm.at[b, blk, 0], ids_v)


    ssems = (ssem0, ssem1)
    csems = (csem0, csem1)
    gsems = (gsem0, gsem1)
    wsems = (wsem0, wsem1)

    def spc_read(c, buf):
        srow = pl.multiple_of(c * P1S, P1S)
        pltpu.async_copy(
            species_hbm.at[blk, pl.ds(srow, P1S)], spc_v.at[buf], ssems[buf])

    def spc_wait(buf):
        pltpu.make_async_copy(
            species_hbm.at[blk, pl.ds(0, P1S)], spc_v.at[buf],
            ssems[buf]).wait()

    def c_write(c, buf):
        crow = pl.multiple_of(cbase + c * P1R, 8)
        pltpu.async_copy(comb_v.at[buf], c_hbm.at[pl.ds(crow, P1R)],
                         csems[buf])

    def c_wait(buf):
        pltpu.make_async_copy(
            comb_v.at[buf], c_hbm.at[pl.ds(cbase, P1R)], csems[buf]).wait()

    def p1_compute(buf, n_s):
        for si in range(n_s):
            for j in range(NUM_STATES):
                for k in range(HS):
                    sl = pl.ds(k * L, L)
                    comb_v[buf, si * NUM_STATES + j, sl] = (
                        state_v[j, sl] + spc_v[buf, si, sl])

    spc_read(0, 0)

    def p1_body(p, carry):
        for buf in range(2):
            c = 2 * p + buf
            spc_read(c + 1, 1 - buf)
            spc_wait(buf)

            @pl.when(p > 0)
            def _cw():
                c_wait(buf)

            p1_compute(buf, P1S)
            c_write(c, buf)
        return carry

    lax.fori_loop(0, (P1N - 1) // 2, p1_body, 0)
    spc_wait(0)
    c_wait(0)
    p1_compute(0, P1S)
    c_write(P1N - 1, 0)
    pltpu.async_copy(
        species_hbm.at[blk, pl.ds(P1N * P1S, P1T)],
        spc_v.at[1, pl.ds(0, P1T)], ssem1)
    pltpu.make_async_copy(
        species_hbm.at[blk, pl.ds(0, P1T)], spc_v.at[1, pl.ds(0, P1T)],
        ssem1).wait()
    c_wait(1)
    p1_compute(1, P1T)
    c_write(P1N, 1)
    for buf in range(2):
        c_wait(buf)

    viota = lax.iota(jnp.int32, L)
    cross8 = jnp.where(viota >= 8, NUM_STATES, 0).astype(jnp.int32)
    cross16 = jnp.where(viota >= L, NUM_STATES, 0).astype(jnp.int32)
    bvec = jnp.zeros((L,), jnp.int32) + cbase

    def idx_body(m, carry):
        base = bvec + 6 * m
        sl0 = pl.ds((3 * m) * L, L)
        idxf_v[sl0] = ids_v[sl0] + base
        sl1 = pl.ds((3 * m + 1) * L, L)
        idxf_v[sl1] = ids_v[sl1] + cross8 + base
        sl2 = pl.ds((3 * m + 2) * L, L)
        idxf_v[sl2] = ids_v[sl2] + cross16 + (base + NUM_STATES)
        return carry

    lax.fori_loop(0, RT // (3 * L), idx_body, 0)

    def gather(cc, buf):
        row = pl.multiple_of(cc * GN, 8)
        pltpu.async_copy(
            c_hbm.at[idxf_v.at[pl.ds(row, GN)]], stage_v.at[buf],
            gsems[buf])

    def gather_wait(buf):
        pltpu.make_async_copy(
            c_hbm.at[idxf_v.at[pl.ds(0, GN)]], stage_v.at[buf],
            gsems[buf]).wait()

    def out_write(cc, buf):
        orow = pl.multiple_of(obase + cc * GN, 8)
        pltpu.async_copy(
            stage_v.at[buf], out_hbm.at[pl.ds(orow, GN)], wsems[buf])

    def out_wait(buf):
        pltpu.make_async_copy(
            stage_v.at[buf], out_hbm.at[pl.ds(obase, GN)], wsems[buf]).wait()

    gather(0, 0)

    def p2_body(p, carry):
        for buf in range(2):
            cc = 2 * p + buf

            if buf == 0:
                @pl.when(p > 0)
                def _ww():
                    out_wait(1)
            else:
                out_wait(0)

            gather(cc + 1, 1 - buf)
            gather_wait(buf)
            out_write(cc, buf)
        return carry

    lax.fori_loop(0, (P2N - 2) // 2, p2_body, 0)
    out_wait(1)
    gather(P2N - 1, 1)
    gather_wait(0)
    out_write(P2N - 2, 0)

    out_wait(0)
    pltpu.async_copy(
        c_hbm.at[idxf_v.at[pl.ds(P2N * GN, GT)]],
        stage_v.at[0, pl.ds(0, GT)], gsem0)
    gather_wait(1)
    out_write(P2N - 1, 1)

    pltpu.make_async_copy(
        c_hbm.at[idxf_v.at[pl.ds(0, GT)]], stage_v.at[0, pl.ds(0, GT)],
        gsem0).wait()
    pltpu.async_copy(
        stage_v.at[0, pl.ds(0, GT)],
        out_hbm.at[pl.ds(obase + P2N * GN, GT)], wsem0)
    pltpu.make_async_copy(
        stage_v.at[0, pl.ds(0, GT)], out_hbm.at[pl.ds(obase, GT)],
        wsem0).wait()
    out_wait(1)


def kernel(input_ids, state_table, species_table):
    ids4 = input_ids.reshape(B, NBLK, 1, SPW * T)
    species3 = species_table.reshape(NBLK, SPW, H)
    mesh = plsc.VectorSubcoreMesh(core_axis_name="c", subcore_axis_name="s")
    f = functools.partial(
        pl.kernel,
        mesh=mesh,
        out_type=(
            jax.ShapeDtypeStruct((B * S * T, H), jnp.float32),
            jax.ShapeDtypeStruct((NW * CPW, H), jnp.float32),
        ),
        scratch_types=[
            pltpu.VMEM((NUM_STATES, H), jnp.float32),
            pltpu.VMEM((SPW * T,), jnp.int32),
            pltpu.VMEM((2, P1S, H), jnp.float32),
            pltpu.VMEM((2, P1R, H), jnp.float32),
            pltpu.VMEM((2, GN, H), jnp.float32),
            pltpu.VMEM((RT,), jnp.int32),
            pltpu.SemaphoreType.DMA,
            pltpu.SemaphoreType.DMA,
            pltpu.SemaphoreType.DMA,
            pltpu.SemaphoreType.DMA,
            pltpu.SemaphoreType.DMA,
            pltpu.SemaphoreType.DMA,
            pltpu.SemaphoreType.DMA,
            pltpu.SemaphoreType.DMA,
        ],
    )(_sc_body)
    out2, _ = f(ids4, state_table, species3)
    return out2.reshape(B, S, T, H)
